# Initial kernel scaffold; baseline (speedup 1.0000x reference)
import jax
import jax.numpy as jnp
from jax.experimental import pallas as pl


def kernel(x_text, x_graph, edge_index, edge_attr, place_node, Wq, bq, Wk, bk, Wv, bv, Ws, bs, Wl, bl, Wl1, bl1, Wt, bt, Wt1, bt1):
    raise NotImplementedError("write your pallas kernel here")


# speedup vs baseline: 4.9156x; 4.9156x over previous
import jax
import jax.numpy as jnp
from jax import lax
from jax.experimental import pallas as pl
from jax.experimental.pallas import tpu as pltpu
from jax.experimental.pallas import tpu_sc as plsc

D = 1536
HID = 128
N = 10000
E = 320000
B = 4096
FF = 4096

NC = 2
NS = 16
NW = NC * NS
EPW = E // NW
C = 40
CHUNKS = EPW // C
NPAD = 10240
RPT = NPAD // NS
ACC_W = HID + 16
SCALE = 1.0 / (HID ** 0.5)


def _scatter_add_rows(src_rows, acc_sh, idx):
    pltpu.sync_copy(src_rows, acc_sh.at[idx], add=True)


def _sc_edge_body(q_hbm, kv_hbm, src_hbm, dst_hbm, num_hbm, den_hbm,
                  src_v, dst_v, q_rows, kv_rows, wv_rows, red_v, den_t,
                  acc_sh, sem):
    c = lax.axis_index("c")
    s = lax.axis_index("s")
    wid = c * NS + s

    zero16 = jnp.zeros((16,), jnp.float32)

    def zero_row(e, carry):
        for j in range(HID // 16):
            wv_rows[e, pl.ds(16 * j, 16)] = zero16
        return carry

    lax.fori_loop(0, C, zero_row, 0)

    def zero_den(i, carry):
        den_t[pl.ds(16 * i, 16)] = zero16
        return carry

    lax.fori_loop(0, NPAD // 16, zero_den, 0)

    for off in range(0, RPT, C):
        n = min(C, RPT - off)
        pltpu.sync_copy(wv_rows.at[pl.ds(0, n)],
                        acc_sh.at[pl.ds(s * RPT + off, n)])
    plsc.subcore_barrier()

    lane = lax.iota(jnp.int32, 16)
    lane0 = lane == 0

    def edge_body(e, carry):
        acc = q_rows[e, pl.ds(0, 16)] * kv_rows[e, pl.ds(0, 16)]
        for j in range(1, HID // 16):
            acc = acc + (q_rows[e, pl.ds(16 * j, 16)]
                         * kv_rows[e, pl.ds(16 * j, 16)])
        for sh in (8, 4, 2, 1):
            red_v[:] = acc
            idx = lane ^ sh
            acc = acc + plsc.load_gather(red_v, [idx])
        w16 = jnp.exp(acc * SCALE)
        for j in range(HID // 16):
            wv_rows[e, pl.ds(16 * j, 16)] = (
                kv_rows[e, pl.ds(HID + 16 * j, 16)] * w16)
        dst_spl = plsc.load_gather(dst_v, [lax.broadcast(e, (16,))])
        plsc.addupdate_scatter(den_t, [dst_spl], w16, mask=lane0)
        return carry

    def chunk_body(t, carry):
        base = wid * EPW + t * C
        pltpu.sync_copy(src_hbm.at[pl.ds(base, C)], src_v)
        pltpu.sync_copy(dst_hbm.at[pl.ds(base, C)], dst_v)
        cp_kv = pltpu.async_copy(kv_hbm.at[src_v], kv_rows, sem)
        cp_q = pltpu.async_copy(q_hbm.at[dst_v], q_rows, sem)
        cp_kv.wait()
        cp_q.wait()
        lax.fori_loop(0, C, edge_body, 0)
        _scatter_add_rows(wv_rows, acc_sh, dst_v)
        return carry

    lax.fori_loop(0, CHUNKS, chunk_body, 0)
    pltpu.sync_copy(den_t, den_hbm.at[wid])
    plsc.subcore_barrier()
    pltpu.sync_copy(acc_sh.at[pl.ds(s * RPT, RPT)],
                    num_hbm.at[c, pl.ds(s * RPT, RPT)])


def _sc_edge_aggregate(q, kv, src, dst):
    mesh = plsc.VectorSubcoreMesh(core_axis_name="c", subcore_axis_name="s",
                                  num_cores=NC, num_subcores=NS)
    kern = pl.kernel(
        _sc_edge_body,
        out_type=(
            jax.ShapeDtypeStruct((NC, NPAD, HID), jnp.float32),
            jax.ShapeDtypeStruct((NW, NPAD), jnp.float32),
        ),
        mesh=mesh,
        compiler_params=pltpu.CompilerParams(needs_layout_passes=False,
                                             use_tc_tiling_on_sc=False),
        scratch_types=[
            pltpu.VMEM((C,), jnp.int32),
            pltpu.VMEM((C,), jnp.int32),
            pltpu.VMEM((C, HID), jnp.float32),
            pltpu.VMEM((C, 2 * HID), jnp.float32),
            pltpu.VMEM((C, HID), jnp.float32),
            pltpu.VMEM((16,), jnp.float32),
            pltpu.VMEM((NPAD,), jnp.float32),
            pltpu.VMEM_SHARED((NPAD, HID), jnp.float32),
            pltpu.SemaphoreType.DMA,
        ],
    )
    return kern(q, kv, src, dst)


def _proj_body(x_ref, w_ref, b_ref, q_ref, kv_ref, skip_ref):
    y = jnp.dot(x_ref[:], w_ref[:], preferred_element_type=jnp.float32)
    y = y + b_ref[:]
    q_ref[:] = y[:, :HID]
    kv_ref[:] = y[:, HID:3 * HID]
    skip_ref[:] = y[:, 3 * HID:]


def _qkvs_projection(x_graph, w_cat_t, b_cat):
    rows = 1024
    grid = (NPAD // rows,)
    return pl.pallas_call(
        _proj_body,
        grid=grid,
        in_specs=[
            pl.BlockSpec((rows, D), lambda i: (i, 0)),
            pl.BlockSpec((D, 4 * HID), lambda i: (0, 0)),
            pl.BlockSpec((1, 4 * HID), lambda i: (0, 0)),
        ],
        out_specs=[
            pl.BlockSpec((rows, HID), lambda i: (i, 0)),
            pl.BlockSpec((rows, 2 * HID), lambda i: (i, 0)),
            pl.BlockSpec((rows, HID), lambda i: (i, 0)),
        ],
        out_shape=[
            jax.ShapeDtypeStruct((NPAD, HID), jnp.float32),
            jax.ShapeDtypeStruct((NPAD, 2 * HID), jnp.float32),
            jax.ShapeDtypeStruct((NPAD, HID), jnp.float32),
        ],
    )(x_graph, w_cat_t, b_cat)


def _mlp_body(num_ref, den_ref, skip_ref, wl_ref, bl_ref, wl1_ref, bl1_ref,
              out_ref, acc_ref):
    i = pl.program_id(0)
    num = num_ref[0] + num_ref[1]
    ones = jnp.ones((NW, 1), jnp.float32)
    den = lax.dot_general(den_ref[:], ones, (((0,), (0,)), ((), ())),
                          preferred_element_type=jnp.float32)
    h = jnp.maximum(num / (den + 1e-16) + skip_ref[:], 0.0)
    y = jnp.dot(h, wl_ref[:], preferred_element_type=jnp.float32) + bl_ref[:]
    y = jnp.maximum(y, 0.0)
    z = jnp.dot(y, wl1_ref[:], preferred_element_type=jnp.float32) + bl1_ref[:]
    z = jnp.maximum(z, 0.0)
    rows = z.shape[0]
    row_id = i * rows + lax.broadcasted_iota(jnp.int32, (rows, 1), 0)
    z = jnp.where(row_id < N, z, 0.0)
    part = jnp.sum(z, axis=0, keepdims=True)

    @pl.when(i == 0)
    def _():
        acc_ref[:] = jnp.zeros_like(acc_ref)

    acc_ref[:] += part

    @pl.when(i == pl.num_programs(0) - 1)
    def _():
        out_ref[:] = acc_ref[:]


def _graph_mlp(num, den, skip, wl_t, bl, wl1_t, bl1):
    rows = 512
    grid = (NPAD // rows,)
    out = pl.pallas_call(
        _mlp_body,
        grid=grid,
        in_specs=[
            pl.BlockSpec((NC, rows, HID), lambda i: (0, i, 0)),
            pl.BlockSpec((NW, rows), lambda i: (0, i)),
            pl.BlockSpec((rows, HID), lambda i: (i, 0)),
            pl.BlockSpec((HID, FF), lambda i: (0, 0)),
            pl.BlockSpec((1, FF), lambda i: (0, 0)),
            pl.BlockSpec((FF, D), lambda i: (0, 0)),
            pl.BlockSpec((1, D), lambda i: (0, 0)),
        ],
        out_specs=pl.BlockSpec((1, D), lambda i: (0, 0)),
        out_shape=jax.ShapeDtypeStruct((1, D), jnp.float32),
        scratch_shapes=[pltpu.VMEM((1, D), jnp.float32)],
    )(num, den, skip, wl_t, bl, wl1_t, bl1)
    return out


def _text_body(x_ref, wt_ref, bt_ref, wt1_ref, bt1_ref, out_ref):
    t = jnp.dot(x_ref[:], wt_ref[:], preferred_element_type=jnp.float32)
    t = jnp.maximum(t + bt_ref[:], 0.0)
    o = jnp.dot(t, wt1_ref[:], preferred_element_type=jnp.float32)
    out_ref[:] = jnp.maximum(o + bt1_ref[:], 0.0)


def _text_mlp(x_text, wt_t, bt, wt1_t, bt1):
    rows = 512
    grid = (B // rows,)
    return pl.pallas_call(
        _text_body,
        grid=grid,
        in_specs=[
            pl.BlockSpec((rows, D), lambda i: (i, 0)),
            pl.BlockSpec((D, HID), lambda i: (0, 0)),
            pl.BlockSpec((1, HID), lambda i: (0, 0)),
            pl.BlockSpec((HID, D), lambda i: (0, 0)),
            pl.BlockSpec((1, D), lambda i: (0, 0)),
        ],
        out_specs=pl.BlockSpec((rows, D), lambda i: (i, 0)),
        out_shape=jax.ShapeDtypeStruct((B, D), jnp.float32),
    )(x_text, wt_t, bt, wt1_t, bt1)


@jax.jit
def kernel(x_text, x_graph, edge_index, edge_attr, place_node,
           Wq, bq, Wk, bk, Wv, bv, Ws, bs,
           Wl, bl, Wl1, bl1, Wt, bt, Wt1, bt1):
    del edge_attr, place_node
    w_cat_t = jnp.concatenate([Wq, Wk, Wv, Ws], axis=0).T
    b_cat = jnp.concatenate([bq, bk, bv, bs]).reshape(1, 4 * HID)

    xg_pad = jnp.pad(x_graph, ((0, NPAD - N), (0, 0)))
    q, kv, skip = _qkvs_projection(xg_pad, w_cat_t, b_cat)

    src = edge_index[0]
    dst = edge_index[1]
    num, den = _sc_edge_aggregate(q, kv, src, dst)

    xt = _text_mlp(x_text, Wt.T, bt.reshape(1, HID), Wt1.T, bt1.reshape(1, D))

    xg = _graph_mlp(num, den, skip, Wl.T, bl.reshape(1, FF),
                    Wl1.T, bl1.reshape(1, D))
    return (xt, xg.reshape(D))


# speedup vs baseline: 9.1758x; 1.8667x over previous
import jax
import jax.numpy as jnp
from jax import lax
from jax.experimental import pallas as pl
from jax.experimental.pallas import tpu as pltpu
from jax.experimental.pallas import tpu_sc as plsc

D = 1536
HID = 128
N = 10000
E = 320000
B = 4096
FF = 4096

NC = 2
NS = 16
NW = NC * NS
EPW = E // NW
C = 40
CHUNKS = EPW // C
NPAD = 10240
RPT = NPAD // NS
ACC_W = HID + 16
SCALE = 1.0 / (HID ** 0.5)


def _sc_edge_body(q_hbm, k_hbm, v_hbm, src_hbm, dst_hbm, num_hbm, den_hbm,
                  src_i, dst_i, dst_sc, q_rows, k_rows, v_rows, red_v, den_t,
                  acc_sh, isem0, isem1, gsem0, gsem1, ssem0, ssem1):
    c = lax.axis_index("c")
    s = lax.axis_index("s")
    wid = c * NS + s
    ebase = wid * EPW

    isems = (isem0, isem1)
    gsems = (gsem0, gsem1)
    ssems = (ssem0, ssem1)

    zero16 = jnp.zeros((16,), jnp.float32)

    def zero_row(e, carry):
        for j in range(HID // 16):
            v_rows[0, e, pl.ds(16 * j, 16)] = zero16
        return carry

    lax.fori_loop(0, C, zero_row, 0)

    def zero_den(i, carry):
        den_t[pl.ds(16 * i, 16)] = zero16
        return carry

    lax.fori_loop(0, NPAD // 16, zero_den, 0)

    for off in range(0, RPT, C):
        n = min(C, RPT - off)
        pltpu.sync_copy(v_rows.at[0, pl.ds(0, n)],
                        acc_sh.at[pl.ds(s * RPT + off, n)])
    plsc.subcore_barrier()

    lane = lax.iota(jnp.int32, 16)
    lane0 = lane == 0

    def issue_idx(t, b):
        pltpu.async_copy(src_hbm.at[pl.ds(ebase + t * C, C)],
                         src_i.at[b], isems[b])
        pltpu.async_copy(dst_hbm.at[pl.ds(ebase + t * C, C)],
                         dst_i.at[b], isems[b])

    def wait_idx(b):
        pltpu.make_async_copy(src_hbm.at[pl.ds(0, C)], src_i.at[b],
                              isems[b]).wait()
        pltpu.make_async_copy(dst_hbm.at[pl.ds(0, C)], dst_i.at[b],
                              isems[b]).wait()

    def issue_gathers(b):
        pltpu.async_copy(q_hbm.at[dst_i.at[b]], q_rows.at[b], gsems[b])
        pltpu.async_copy(k_hbm.at[src_i.at[b]], k_rows.at[b], gsems[b])
        pltpu.async_copy(v_hbm.at[src_i.at[b]], v_rows.at[b], gsems[b])

    def wait_gathers(b):
        pltpu.make_async_copy(q_hbm.at[pl.ds(0, C)], q_rows.at[b],
                              gsems[b]).wait()
        pltpu.make_async_copy(k_hbm.at[pl.ds(0, C)], k_rows.at[b],
                              gsems[b]).wait()
        pltpu.make_async_copy(v_hbm.at[pl.ds(0, C)], v_rows.at[b],
                              gsems[b]).wait()

    def copy_idx(b):
        for off in (0, 16, C - 16):
            dst_sc[b, pl.ds(off, 16)] = dst_i[b, pl.ds(off, 16)]

    def issue_scatter(b):
        pltpu.async_copy(v_rows.at[b], acc_sh.at[dst_sc.at[b]], ssems[b],
                         add=True)

    def wait_scatter(b):
        pltpu.make_async_copy(v_rows.at[b], acc_sh.at[pl.ds(0, C)],
                              ssems[b]).wait()

    def compute_chunk(b):
        def edge_body(e, carry):
            acc = q_rows[b, e, pl.ds(0, 16)] * k_rows[b, e, pl.ds(0, 16)]
            for j in range(1, HID // 16):
                acc = acc + (q_rows[b, e, pl.ds(16 * j, 16)]
                             * k_rows[b, e, pl.ds(16 * j, 16)])
            for sh in (8, 4, 2, 1):
                red_v[:] = acc
                acc = acc + plsc.load_gather(red_v, [lane ^ sh])
            w16 = jnp.exp(acc * SCALE)
            for j in range(HID // 16):
                sl = pl.ds(16 * j, 16)
                v_rows[b, e, sl] = v_rows[b, e, sl] * w16
            dst_spl = plsc.load_gather(dst_sc.at[b], [lax.broadcast(e, (16,))])
            plsc.addupdate_scatter(den_t, [dst_spl], w16, mask=lane0)
            return carry

        lax.fori_loop(0, C, edge_body, 0)

    issue_idx(0, 0)
    issue_idx(1, 1)
    wait_idx(0)
    issue_gathers(0)

    def pipe_body(i, carry):
        t0 = 2 * i
        t1 = 2 * i + 1
        wait_idx(1)

        @pl.when(t0 > 0)
        def _():
            wait_scatter(1)

        issue_gathers(1)
        wait_gathers(0)
        copy_idx(0)

        @pl.when(t0 + 2 < CHUNKS)
        def _():
            issue_idx(t0 + 2, 0)

        compute_chunk(0)
        issue_scatter(0)

        @pl.when(t1 + 1 < CHUNKS)
        def _():
            wait_idx(0)
            wait_scatter(0)
            issue_gathers(0)

        wait_gathers(1)
        copy_idx(1)

        @pl.when(t1 + 2 < CHUNKS)
        def _():
            issue_idx(t1 + 2, 1)

        compute_chunk(1)
        issue_scatter(1)
        return carry

    lax.fori_loop(0, CHUNKS // 2, pipe_body, 0)
    wait_scatter(0)
    wait_scatter(1)

    pltpu.sync_copy(den_t, den_hbm.at[wid])
    plsc.subcore_barrier()
    pltpu.sync_copy(acc_sh.at[pl.ds(s * RPT, RPT)],
                    num_hbm.at[c, pl.ds(s * RPT, RPT)])


def _sc_edge_aggregate(q, k, v, src, dst):
    mesh = plsc.VectorSubcoreMesh(core_axis_name="c", subcore_axis_name="s",
                                  num_cores=NC, num_subcores=NS)
    kern = pl.kernel(
        _sc_edge_body,
        out_type=(
            jax.ShapeDtypeStruct((NC, NPAD, HID), jnp.float32),
            jax.ShapeDtypeStruct((NW, NPAD), jnp.float32),
        ),
        mesh=mesh,
        compiler_params=pltpu.CompilerParams(needs_layout_passes=False,
                                             use_tc_tiling_on_sc=False),
        scratch_types=[
            pltpu.VMEM((2, C), jnp.int32),
            pltpu.VMEM((2, C), jnp.int32),
            pltpu.VMEM((2, C), jnp.int32),
            pltpu.VMEM((2, C, HID), jnp.float32),
            pltpu.VMEM((2, C, HID), jnp.float32),
            pltpu.VMEM((2, C, HID), jnp.float32),
            pltpu.VMEM((16,), jnp.float32),
            pltpu.VMEM((NPAD,), jnp.float32),
            pltpu.VMEM_SHARED((NPAD, HID), jnp.float32),
            pltpu.SemaphoreType.DMA,
            pltpu.SemaphoreType.DMA,
            pltpu.SemaphoreType.DMA,
            pltpu.SemaphoreType.DMA,
            pltpu.SemaphoreType.DMA,
            pltpu.SemaphoreType.DMA,
        ],
    )
    return kern(q, k, v, src, dst)


def _proj_body(x_ref, w_ref, b_ref, q_ref, k_ref, v_ref, skip_ref):
    y = jnp.dot(x_ref[:], w_ref[:], preferred_element_type=jnp.float32)
    y = y + b_ref[:]
    q_ref[:] = y[:, :HID]
    k_ref[:] = y[:, HID:2 * HID]
    v_ref[:] = y[:, 2 * HID:3 * HID]
    skip_ref[:] = y[:, 3 * HID:]


def _qkvs_projection(x_graph, w_cat_t, b_cat):
    rows = 1024
    grid = (NPAD // rows,)
    return pl.pallas_call(
        _proj_body,
        grid=grid,
        in_specs=[
            pl.BlockSpec((rows, D), lambda i: (i, 0)),
            pl.BlockSpec((D, 4 * HID), lambda i: (0, 0)),
            pl.BlockSpec((1, 4 * HID), lambda i: (0, 0)),
        ],
        out_specs=[
            pl.BlockSpec((rows, HID), lambda i: (i, 0)),
            pl.BlockSpec((rows, HID), lambda i: (i, 0)),
            pl.BlockSpec((rows, HID), lambda i: (i, 0)),
            pl.BlockSpec((rows, HID), lambda i: (i, 0)),
        ],
        out_shape=[
            jax.ShapeDtypeStruct((NPAD, HID), jnp.float32),
            jax.ShapeDtypeStruct((NPAD, HID), jnp.float32),
            jax.ShapeDtypeStruct((NPAD, HID), jnp.float32),
            jax.ShapeDtypeStruct((NPAD, HID), jnp.float32),
        ],
    )(x_graph, w_cat_t, b_cat)


def _mlp_body(num_ref, den_ref, skip_ref, wl_ref, bl_ref, wl1_ref, bl1_ref,
              out_ref, acc_ref):
    i = pl.program_id(0)
    num = num_ref[0] + num_ref[1]
    ones = jnp.ones((NW, 1), jnp.float32)
    den = lax.dot_general(den_ref[:], ones, (((0,), (0,)), ((), ())),
                          preferred_element_type=jnp.float32)
    h = jnp.maximum(num / (den + 1e-16) + skip_ref[:], 0.0)
    y = jnp.dot(h, wl_ref[:], preferred_element_type=jnp.float32) + bl_ref[:]
    y = jnp.maximum(y, 0.0)
    z = jnp.dot(y, wl1_ref[:], preferred_element_type=jnp.float32) + bl1_ref[:]
    z = jnp.maximum(z, 0.0)
    rows = z.shape[0]
    row_id = i * rows + lax.broadcasted_iota(jnp.int32, (rows, 1), 0)
    z = jnp.where(row_id < N, z, 0.0)
    part = jnp.sum(z, axis=0, keepdims=True)

    @pl.when(i == 0)
    def _():
        acc_ref[:] = jnp.zeros_like(acc_ref)

    acc_ref[:] += part

    @pl.when(i == pl.num_programs(0) - 1)
    def _():
        out_ref[:] = acc_ref[:]


def _graph_mlp(num, den, skip, wl_t, bl, wl1_t, bl1):
    rows = 512
    grid = (NPAD // rows,)
    out = pl.pallas_call(
        _mlp_body,
        grid=grid,
        in_specs=[
            pl.BlockSpec((NC, rows, HID), lambda i: (0, i, 0)),
            pl.BlockSpec((NW, rows), lambda i: (0, i)),
            pl.BlockSpec((rows, HID), lambda i: (i, 0)),
            pl.BlockSpec((HID, FF), lambda i: (0, 0)),
            pl.BlockSpec((1, FF), lambda i: (0, 0)),
            pl.BlockSpec((FF, D), lambda i: (0, 0)),
            pl.BlockSpec((1, D), lambda i: (0, 0)),
        ],
        out_specs=pl.BlockSpec((1, D), lambda i: (0, 0)),
        out_shape=jax.ShapeDtypeStruct((1, D), jnp.float32),
        scratch_shapes=[pltpu.VMEM((1, D), jnp.float32)],
    )(num, den, skip, wl_t, bl, wl1_t, bl1)
    return out


def _text_body(x_ref, wt_ref, bt_ref, wt1_ref, bt1_ref, out_ref):
    t = jnp.dot(x_ref[:], wt_ref[:], preferred_element_type=jnp.float32)
    t = jnp.maximum(t + bt_ref[:], 0.0)
    o = jnp.dot(t, wt1_ref[:], preferred_element_type=jnp.float32)
    out_ref[:] = jnp.maximum(o + bt1_ref[:], 0.0)


def _text_mlp(x_text, wt_t, bt, wt1_t, bt1):
    rows = 512
    grid = (B // rows,)
    return pl.pallas_call(
        _text_body,
        grid=grid,
        in_specs=[
            pl.BlockSpec((rows, D), lambda i: (i, 0)),
            pl.BlockSpec((D, HID), lambda i: (0, 0)),
            pl.BlockSpec((1, HID), lambda i: (0, 0)),
            pl.BlockSpec((HID, D), lambda i: (0, 0)),
            pl.BlockSpec((1, D), lambda i: (0, 0)),
        ],
        out_specs=pl.BlockSpec((rows, D), lambda i: (i, 0)),
        out_shape=jax.ShapeDtypeStruct((B, D), jnp.float32),
    )(x_text, wt_t, bt, wt1_t, bt1)


@jax.jit
def kernel(x_text, x_graph, edge_index, edge_attr, place_node,
           Wq, bq, Wk, bk, Wv, bv, Ws, bs,
           Wl, bl, Wl1, bl1, Wt, bt, Wt1, bt1):
    del edge_attr, place_node
    w_cat_t = jnp.concatenate([Wq, Wk, Wv, Ws], axis=0).T
    b_cat = jnp.concatenate([bq, bk, bv, bs]).reshape(1, 4 * HID)

    xg_pad = jnp.pad(x_graph, ((0, NPAD - N), (0, 0)))
    q, k, v, skip = _qkvs_projection(xg_pad, w_cat_t, b_cat)

    src = edge_index[0]
    dst = edge_index[1]
    num, den = _sc_edge_aggregate(q, k, v, src, dst)

    xt = _text_mlp(x_text, Wt.T, bt.reshape(1, HID), Wt1.T, bt1.reshape(1, D))

    xg = _graph_mlp(num, den, skip, Wl.T, bl.reshape(1, FF),
                    Wl1.T, bl1.reshape(1, D))
    return (xt, xg.reshape(D))


# speedup vs baseline: 9.4559x; 1.0305x over previous
import jax
import jax.numpy as jnp
from jax import lax
from jax.experimental import pallas as pl
from jax.experimental.pallas import tpu as pltpu
from jax.experimental.pallas import tpu_sc as plsc

D = 1536
HID = 128
N = 10000
E = 320000
B = 4096
FF = 4096

NC = 2
NS = 16
NW = NC * NS
EPW = E // NW
C = 40
CHUNKS = EPW // C
NPAD = 10240
RPT = NPAD // NS
ACC_W = HID + 16
SCALE = 1.0 / (HID ** 0.5)


def _sc_edge_body(q_hbm, k_hbm, v_hbm, src_hbm, dst_hbm, num_hbm, den_hbm,
                  src_i, dst_i, dst_sc, q_rows, k_rows, v_rows, den_rows,
                  red_v, acc_sh, den_sh,
                  isem0, isem1, gsem0, gsem1, ssem0, ssem1):
    c = lax.axis_index("c")
    s = lax.axis_index("s")
    wid = c * NS + s
    ebase = wid * EPW

    isems = (isem0, isem1)
    gsems = (gsem0, gsem1)
    ssems = (ssem0, ssem1)

    zero16 = jnp.zeros((16,), jnp.float32)

    def zero_row(e, carry):
        for j in range(HID // 16):
            v_rows[0, e, pl.ds(16 * j, 16)] = zero16
        den_rows[0, e, pl.ds(0, 16)] = zero16
        return carry

    lax.fori_loop(0, C, zero_row, 0)

    for off in range(0, RPT, C):
        n = min(C, RPT - off)
        pltpu.sync_copy(v_rows.at[0, pl.ds(0, n)],
                        acc_sh.at[pl.ds(s * RPT + off, n)])
        pltpu.sync_copy(den_rows.at[0, pl.ds(0, n)],
                        den_sh.at[pl.ds(s * RPT + off, n)])
    plsc.subcore_barrier()

    lane = lax.iota(jnp.int32, 16)
    den_mask = jnp.where(lane == 0, 1.0, 0.0).astype(jnp.float32)

    def issue_idx(t, b):
        pltpu.async_copy(src_hbm.at[pl.ds(ebase + t * C, C)],
                         src_i.at[b], isems[b])
        pltpu.async_copy(dst_hbm.at[pl.ds(ebase + t * C, C)],
                         dst_i.at[b], isems[b])

    def wait_idx(b):
        pltpu.make_async_copy(src_hbm.at[pl.ds(0, C)], src_i.at[b],
                              isems[b]).wait()
        pltpu.make_async_copy(dst_hbm.at[pl.ds(0, C)], dst_i.at[b],
                              isems[b]).wait()

    def issue_gathers(b):
        pltpu.async_copy(q_hbm.at[dst_i.at[b]], q_rows.at[b], gsems[b])
        pltpu.async_copy(k_hbm.at[src_i.at[b]], k_rows.at[b], gsems[b])
        pltpu.async_copy(v_hbm.at[src_i.at[b]], v_rows.at[b], gsems[b])

    def wait_gathers(b):
        pltpu.make_async_copy(q_hbm.at[pl.ds(0, C)], q_rows.at[b],
                              gsems[b]).wait()
        pltpu.make_async_copy(k_hbm.at[pl.ds(0, C)], k_rows.at[b],
                              gsems[b]).wait()
        pltpu.make_async_copy(v_hbm.at[pl.ds(0, C)], v_rows.at[b],
                              gsems[b]).wait()

    def copy_idx(b):
        for off in (0, 16, C - 16):
            dst_sc[b, pl.ds(off, 16)] = dst_i[b, pl.ds(off, 16)]

    def issue_scatter(b):
        pltpu.async_copy(v_rows.at[b], acc_sh.at[dst_sc.at[b]], ssems[b],
                         add=True)
        pltpu.async_copy(den_rows.at[b], den_sh.at[dst_sc.at[b]], ssems[b],
                         add=True)

    def wait_scatter(b):
        pltpu.make_async_copy(v_rows.at[b], acc_sh.at[pl.ds(0, C)],
                              ssems[b]).wait()
        pltpu.make_async_copy(den_rows.at[b], den_sh.at[pl.ds(0, C)],
                              ssems[b]).wait()

    def compute_chunk(b):
        def edge_body(e, carry):
            acc = q_rows[b, e, pl.ds(0, 16)] * k_rows[b, e, pl.ds(0, 16)]
            for j in range(1, HID // 16):
                acc = acc + (q_rows[b, e, pl.ds(16 * j, 16)]
                             * k_rows[b, e, pl.ds(16 * j, 16)])
            for sh in (8, 4, 2, 1):
                red_v[:] = acc
                acc = acc + plsc.load_gather(red_v, [lane ^ sh])
            w16 = jnp.exp(acc * SCALE)
            for j in range(HID // 16):
                sl = pl.ds(16 * j, 16)
                v_rows[b, e, sl] = v_rows[b, e, sl] * w16
            den_rows[b, e, pl.ds(0, 16)] = w16 * den_mask
            return carry

        lax.fori_loop(0, C, edge_body, 0)

    issue_idx(0, 0)
    issue_idx(1, 1)
    wait_idx(0)
    issue_gathers(0)

    def pipe_body(i, carry):
        t0 = 2 * i
        t1 = 2 * i + 1
        wait_idx(1)

        @pl.when(t0 > 0)
        def _():
            wait_scatter(1)

        issue_gathers(1)
        wait_gathers(0)
        copy_idx(0)

        @pl.when(t0 + 2 < CHUNKS)
        def _():
            issue_idx(t0 + 2, 0)

        compute_chunk(0)
        issue_scatter(0)

        @pl.when(t1 + 1 < CHUNKS)
        def _():
            wait_idx(0)
            wait_scatter(0)
            issue_gathers(0)

        wait_gathers(1)
        copy_idx(1)

        @pl.when(t1 + 2 < CHUNKS)
        def _():
            issue_idx(t1 + 2, 1)

        compute_chunk(1)
        issue_scatter(1)
        return carry

    lax.fori_loop(0, CHUNKS // 2, pipe_body, 0)
    wait_scatter(0)
    wait_scatter(1)
    plsc.subcore_barrier()
    pltpu.sync_copy(acc_sh.at[pl.ds(s * RPT, RPT)],
                    num_hbm.at[c, pl.ds(s * RPT, RPT)])
    pltpu.sync_copy(den_sh.at[pl.ds(s * RPT, RPT)],
                    den_hbm.at[c, pl.ds(s * RPT, RPT)])


def _sc_edge_aggregate(q, k, v, src, dst):
    mesh = plsc.VectorSubcoreMesh(core_axis_name="c", subcore_axis_name="s",
                                  num_cores=NC, num_subcores=NS)
    kern = pl.kernel(
        _sc_edge_body,
        out_type=(
            jax.ShapeDtypeStruct((NC, NPAD, HID), jnp.float32),
            jax.ShapeDtypeStruct((NC, NPAD, 16), jnp.float32),
        ),
        mesh=mesh,
        compiler_params=pltpu.CompilerParams(needs_layout_passes=False,
                                             use_tc_tiling_on_sc=False),
        scratch_types=[
            pltpu.VMEM((2, C), jnp.int32),
            pltpu.VMEM((2, C), jnp.int32),
            pltpu.VMEM((2, C), jnp.int32),
            pltpu.VMEM((2, C, HID), jnp.float32),
            pltpu.VMEM((2, C, HID), jnp.float32),
            pltpu.VMEM((2, C, HID), jnp.float32),
            pltpu.VMEM((2, C, 16), jnp.float32),
            pltpu.VMEM((16,), jnp.float32),
            pltpu.VMEM_SHARED((NPAD, HID), jnp.float32),
            pltpu.VMEM_SHARED((NPAD, 16), jnp.float32),
            pltpu.SemaphoreType.DMA,
            pltpu.SemaphoreType.DMA,
            pltpu.SemaphoreType.DMA,
            pltpu.SemaphoreType.DMA,
            pltpu.SemaphoreType.DMA,
            pltpu.SemaphoreType.DMA,
        ],
    )
    return kern(q, k, v, src, dst)


def _proj_body(x_ref, w_ref, b_ref, q_ref, k_ref, v_ref, skip_ref):
    y = jnp.dot(x_ref[:], w_ref[:], preferred_element_type=jnp.float32)
    y = y + b_ref[:]
    q_ref[:] = y[:, :HID]
    k_ref[:] = y[:, HID:2 * HID]
    v_ref[:] = y[:, 2 * HID:3 * HID]
    skip_ref[:] = y[:, 3 * HID:]


def _qkvs_projection(x_graph, w_cat_t, b_cat):
    rows = 1024
    grid = (NPAD // rows,)
    return pl.pallas_call(
        _proj_body,
        grid=grid,
        in_specs=[
            pl.BlockSpec((rows, D), lambda i: (i, 0)),
            pl.BlockSpec((D, 4 * HID), lambda i: (0, 0)),
            pl.BlockSpec((1, 4 * HID), lambda i: (0, 0)),
        ],
        out_specs=[
            pl.BlockSpec((rows, HID), lambda i: (i, 0)),
            pl.BlockSpec((rows, HID), lambda i: (i, 0)),
            pl.BlockSpec((rows, HID), lambda i: (i, 0)),
            pl.BlockSpec((rows, HID), lambda i: (i, 0)),
        ],
        out_shape=[
            jax.ShapeDtypeStruct((NPAD, HID), jnp.float32),
            jax.ShapeDtypeStruct((NPAD, HID), jnp.float32),
            jax.ShapeDtypeStruct((NPAD, HID), jnp.float32),
            jax.ShapeDtypeStruct((NPAD, HID), jnp.float32),
        ],
    )(x_graph, w_cat_t, b_cat)


def _mlp_body(num_ref, den_ref, skip_ref, wl_ref, bl_ref, wl1_ref, bl1_ref,
              out_ref, acc_ref):
    i = pl.program_id(0)
    num = num_ref[0] + num_ref[1]
    den = den_ref[0, :, 0:1] + den_ref[1, :, 0:1]
    h = jnp.maximum(num / (den + 1e-16) + skip_ref[:], 0.0)
    y = jnp.dot(h, wl_ref[:], preferred_element_type=jnp.float32) + bl_ref[:]
    y = jnp.maximum(y, 0.0)
    z = jnp.dot(y, wl1_ref[:], preferred_element_type=jnp.float32) + bl1_ref[:]
    z = jnp.maximum(z, 0.0)
    rows = z.shape[0]
    row_id = i * rows + lax.broadcasted_iota(jnp.int32, (rows, 1), 0)
    z = jnp.where(row_id < N, z, 0.0)
    part = jnp.sum(z, axis=0, keepdims=True)

    @pl.when(i == 0)
    def _():
        acc_ref[:] = jnp.zeros_like(acc_ref)

    acc_ref[:] += part

    @pl.when(i == pl.num_programs(0) - 1)
    def _():
        out_ref[:] = acc_ref[:]


def _graph_mlp(num, den, skip, wl_t, bl, wl1_t, bl1):
    rows = 512
    grid = (NPAD // rows,)
    out = pl.pallas_call(
        _mlp_body,
        grid=grid,
        in_specs=[
            pl.BlockSpec((NC, rows, HID), lambda i: (0, i, 0)),
            pl.BlockSpec((NC, rows, 16), lambda i: (0, i, 0)),
            pl.BlockSpec((rows, HID), lambda i: (i, 0)),
            pl.BlockSpec((HID, FF), lambda i: (0, 0)),
            pl.BlockSpec((1, FF), lambda i: (0, 0)),
            pl.BlockSpec((FF, D), lambda i: (0, 0)),
            pl.BlockSpec((1, D), lambda i: (0, 0)),
        ],
        out_specs=pl.BlockSpec((1, D), lambda i: (0, 0)),
        out_shape=jax.ShapeDtypeStruct((1, D), jnp.float32),
        scratch_shapes=[pltpu.VMEM((1, D), jnp.float32)],
    )(num, den, skip, wl_t, bl, wl1_t, bl1)
    return out


def _text_body(x_ref, wt_ref, bt_ref, wt1_ref, bt1_ref, out_ref):
    t = jnp.dot(x_ref[:], wt_ref[:], preferred_element_type=jnp.float32)
    t = jnp.maximum(t + bt_ref[:], 0.0)
    o = jnp.dot(t, wt1_ref[:], preferred_element_type=jnp.float32)
    out_ref[:] = jnp.maximum(o + bt1_ref[:], 0.0)


def _text_mlp(x_text, wt_t, bt, wt1_t, bt1):
    rows = 512
    grid = (B // rows,)
    return pl.pallas_call(
        _text_body,
        grid=grid,
        in_specs=[
            pl.BlockSpec((rows, D), lambda i: (i, 0)),
            pl.BlockSpec((D, HID), lambda i: (0, 0)),
            pl.BlockSpec((1, HID), lambda i: (0, 0)),
            pl.BlockSpec((HID, D), lambda i: (0, 0)),
            pl.BlockSpec((1, D), lambda i: (0, 0)),
        ],
        out_specs=pl.BlockSpec((rows, D), lambda i: (i, 0)),
        out_shape=jax.ShapeDtypeStruct((B, D), jnp.float32),
    )(x_text, wt_t, bt, wt1_t, bt1)


@jax.jit
def kernel(x_text, x_graph, edge_index, edge_attr, place_node,
           Wq, bq, Wk, bk, Wv, bv, Ws, bs,
           Wl, bl, Wl1, bl1, Wt, bt, Wt1, bt1):
    del edge_attr, place_node
    w_cat_t = jnp.concatenate([Wq, Wk, Wv, Ws], axis=0).T
    b_cat = jnp.concatenate([bq, bk, bv, bs]).reshape(1, 4 * HID)

    xg_pad = jnp.pad(x_graph, ((0, NPAD - N), (0, 0)))
    q, k, v, skip = _qkvs_projection(xg_pad, w_cat_t, b_cat)

    src = edge_index[0]
    dst = edge_index[1]
    num, den = _sc_edge_aggregate(q, k, v, src, dst)

    xt = _text_mlp(x_text, Wt.T, bt.reshape(1, HID), Wt1.T, bt1.reshape(1, D))

    xg = _graph_mlp(num, den, skip, Wl.T, bl.reshape(1, FF),
                    Wl1.T, bl1.reshape(1, D))
    return (xt, xg.reshape(D))


# speedup vs baseline: 12.5831x; 1.3307x over previous
import jax
import jax.numpy as jnp
from jax import lax
from jax.experimental import pallas as pl
from jax.experimental.pallas import tpu as pltpu
from jax.experimental.pallas import tpu_sc as plsc

D = 1536
HID = 128
N = 10000
E = 320000
B = 4096
FF = 4096

NC = 2
NS = 16
NW = NC * NS
EPW = E // NW
C = 40
UNR = 4
CHUNKS = EPW // C
NPAD = 10240
RPT = NPAD // NS
ACC_W = HID + 16
SCALE = 1.0 / (HID ** 0.5)


def _sc_edge_body(q_hbm, k_hbm, v_hbm, src_hbm, dst_hbm, num_hbm, den_hbm,
                  src_i, dst_i, dst_sc, q_rows, k_rows, v_rows, den_rows,
                  red_v, acc_sh, den_sh,
                  isem0, isem1, gsem0, gsem1, ssem0, ssem1):
    c = lax.axis_index("c")
    s = lax.axis_index("s")
    wid = c * NS + s
    ebase = wid * EPW

    isems = (isem0, isem1)
    gsems = (gsem0, gsem1)
    ssems = (ssem0, ssem1)

    zero16 = jnp.zeros((16,), jnp.float32)

    def zero_row(e, carry):
        for j in range(HID // 16):
            v_rows[0, e, pl.ds(16 * j, 16)] = zero16
        den_rows[0, e, pl.ds(0, 16)] = zero16
        return carry

    lax.fori_loop(0, C, zero_row, 0)

    for off in range(0, RPT, C):
        n = min(C, RPT - off)
        pltpu.sync_copy(v_rows.at[0, pl.ds(0, n)],
                        acc_sh.at[pl.ds(s * RPT + off, n)])
        pltpu.sync_copy(den_rows.at[0, pl.ds(0, n)],
                        den_sh.at[pl.ds(s * RPT + off, n)])
    plsc.subcore_barrier()

    lane = lax.iota(jnp.int32, 16)
    den_mask = jnp.where(lane == 0, 1.0, 0.0).astype(jnp.float32)

    def issue_idx(t, b):
        pltpu.async_copy(src_hbm.at[pl.ds(ebase + t * C, C)],
                         src_i.at[b], isems[b])
        pltpu.async_copy(dst_hbm.at[pl.ds(ebase + t * C, C)],
                         dst_i.at[b], isems[b])

    def wait_idx(b):
        pltpu.make_async_copy(src_hbm.at[pl.ds(0, C)], src_i.at[b],
                              isems[b]).wait()
        pltpu.make_async_copy(dst_hbm.at[pl.ds(0, C)], dst_i.at[b],
                              isems[b]).wait()

    def issue_gathers(b):
        pltpu.async_copy(q_hbm.at[dst_i.at[b]], q_rows.at[b], gsems[b])
        pltpu.async_copy(k_hbm.at[src_i.at[b]], k_rows.at[b], gsems[b])
        pltpu.async_copy(v_hbm.at[src_i.at[b]], v_rows.at[b], gsems[b])

    def wait_gathers(b):
        pltpu.make_async_copy(q_hbm.at[pl.ds(0, C)], q_rows.at[b],
                              gsems[b]).wait()
        pltpu.make_async_copy(k_hbm.at[pl.ds(0, C)], k_rows.at[b],
                              gsems[b]).wait()
        pltpu.make_async_copy(v_hbm.at[pl.ds(0, C)], v_rows.at[b],
                              gsems[b]).wait()

    def copy_idx(b):
        for off in (0, 16, C - 16):
            dst_sc[b, pl.ds(off, 16)] = dst_i[b, pl.ds(off, 16)]

    def issue_scatter(b):
        pltpu.async_copy(v_rows.at[b], acc_sh.at[dst_sc.at[b]], ssems[b],
                         add=True)
        pltpu.async_copy(den_rows.at[b], den_sh.at[dst_sc.at[b]], ssems[b],
                         add=True)

    def wait_scatter(b):
        pltpu.make_async_copy(v_rows.at[b], acc_sh.at[pl.ds(0, C)],
                              ssems[b]).wait()
        pltpu.make_async_copy(den_rows.at[b], den_sh.at[pl.ds(0, C)],
                              ssems[b]).wait()

    def compute_chunk(b):
        def edge_group(g, carry):
            e0 = g * UNR
            accs = []
            for u in range(UNR):
                e = e0 + u
                acc = q_rows[b, e, pl.ds(0, 16)] * k_rows[b, e, pl.ds(0, 16)]
                for j in range(1, HID // 16):
                    acc = acc + (q_rows[b, e, pl.ds(16 * j, 16)]
                                 * k_rows[b, e, pl.ds(16 * j, 16)])
                accs.append(acc)
            for sh in (8, 4, 2, 1):
                for u in range(UNR):
                    red_v[u, pl.ds(0, 16)] = accs[u]
                accs = [accs[u] + plsc.load_gather(red_v.at[u], [lane ^ sh])
                        for u in range(UNR)]
            ws = [jnp.exp(accs[u] * SCALE) for u in range(UNR)]
            for u in range(UNR):
                e = e0 + u
                for j in range(HID // 16):
                    sl = pl.ds(16 * j, 16)
                    v_rows[b, e, sl] = v_rows[b, e, sl] * ws[u]
                den_rows[b, e, pl.ds(0, 16)] = ws[u] * den_mask
            return carry

        lax.fori_loop(0, C // UNR, edge_group, 0)

    issue_idx(0, 0)
    issue_idx(1, 1)
    wait_idx(0)
    issue_gathers(0)

    def pipe_body(i, carry):
        t0 = 2 * i
        t1 = 2 * i + 1
        wait_idx(1)

        @pl.when(t0 > 0)
        def _():
            wait_scatter(1)

        issue_gathers(1)
        wait_gathers(0)
        copy_idx(0)

        @pl.when(t0 + 2 < CHUNKS)
        def _():
            issue_idx(t0 + 2, 0)

        compute_chunk(0)
        issue_scatter(0)

        @pl.when(t1 + 1 < CHUNKS)
        def _():
            wait_idx(0)
            wait_scatter(0)
            issue_gathers(0)

        wait_gathers(1)
        copy_idx(1)

        @pl.when(t1 + 2 < CHUNKS)
        def _():
            issue_idx(t1 + 2, 1)

        compute_chunk(1)
        issue_scatter(1)
        return carry

    lax.fori_loop(0, CHUNKS // 2, pipe_body, 0)
    wait_scatter(0)
    wait_scatter(1)
    plsc.subcore_barrier()
    pltpu.sync_copy(acc_sh.at[pl.ds(s * RPT, RPT)],
                    num_hbm.at[c, pl.ds(s * RPT, RPT)])
    pltpu.sync_copy(den_sh.at[pl.ds(s * RPT, RPT)],
                    den_hbm.at[c, pl.ds(s * RPT, RPT)])


def _sc_edge_aggregate(q, k, v, src, dst):
    mesh = plsc.VectorSubcoreMesh(core_axis_name="c", subcore_axis_name="s",
                                  num_cores=NC, num_subcores=NS)
    kern = pl.kernel(
        _sc_edge_body,
        out_type=(
            jax.ShapeDtypeStruct((NC, NPAD, HID), jnp.float32),
            jax.ShapeDtypeStruct((NC, NPAD, 16), jnp.float32),
        ),
        mesh=mesh,
        compiler_params=pltpu.CompilerParams(needs_layout_passes=False,
                                             use_tc_tiling_on_sc=False),
        scratch_types=[
            pltpu.VMEM((2, C), jnp.int32),
            pltpu.VMEM((2, C), jnp.int32),
            pltpu.VMEM((2, C), jnp.int32),
            pltpu.VMEM((2, C, HID), jnp.float32),
            pltpu.VMEM((2, C, HID), jnp.float32),
            pltpu.VMEM((2, C, HID), jnp.float32),
            pltpu.VMEM((2, C, 16), jnp.float32),
            pltpu.VMEM((UNR, 16), jnp.float32),
            pltpu.VMEM_SHARED((NPAD, HID), jnp.float32),
            pltpu.VMEM_SHARED((NPAD, 16), jnp.float32),
            pltpu.SemaphoreType.DMA,
            pltpu.SemaphoreType.DMA,
            pltpu.SemaphoreType.DMA,
            pltpu.SemaphoreType.DMA,
            pltpu.SemaphoreType.DMA,
            pltpu.SemaphoreType.DMA,
        ],
    )
    return kern(q, k, v, src, dst)


def _proj_body(x_ref, w_ref, b_ref, q_ref, k_ref, v_ref, skip_ref):
    y = jnp.dot(x_ref[:], w_ref[:], preferred_element_type=jnp.float32)
    y = y + b_ref[:]
    q_ref[:] = y[:, :HID]
    k_ref[:] = y[:, HID:2 * HID]
    v_ref[:] = y[:, 2 * HID:3 * HID]
    skip_ref[:] = y[:, 3 * HID:]


def _qkvs_projection(x_graph, w_cat_t, b_cat):
    rows = 1024
    grid = (NPAD // rows,)
    return pl.pallas_call(
        _proj_body,
        grid=grid,
        in_specs=[
            pl.BlockSpec((rows, D), lambda i: (i, 0)),
            pl.BlockSpec((D, 4 * HID), lambda i: (0, 0)),
            pl.BlockSpec((1, 4 * HID), lambda i: (0, 0)),
        ],
        out_specs=[
            pl.BlockSpec((rows, HID), lambda i: (i, 0)),
            pl.BlockSpec((rows, HID), lambda i: (i, 0)),
            pl.BlockSpec((rows, HID), lambda i: (i, 0)),
            pl.BlockSpec((rows, HID), lambda i: (i, 0)),
        ],
        out_shape=[
            jax.ShapeDtypeStruct((NPAD, HID), jnp.float32),
            jax.ShapeDtypeStruct((NPAD, HID), jnp.float32),
            jax.ShapeDtypeStruct((NPAD, HID), jnp.float32),
            jax.ShapeDtypeStruct((NPAD, HID), jnp.float32),
        ],
    )(x_graph, w_cat_t, b_cat)


def _mlp_body(num_ref, den_ref, skip_ref, wl_ref, bl_ref, wl1_ref, bl1_ref,
              out_ref, acc_ref):
    i = pl.program_id(0)
    num = num_ref[0] + num_ref[1]
    den = den_ref[0, :, 0:1] + den_ref[1, :, 0:1]
    h = jnp.maximum(num / (den + 1e-16) + skip_ref[:], 0.0)
    y = jnp.dot(h, wl_ref[:], preferred_element_type=jnp.float32) + bl_ref[:]
    y = jnp.maximum(y, 0.0)
    z = jnp.dot(y, wl1_ref[:], preferred_element_type=jnp.float32) + bl1_ref[:]
    z = jnp.maximum(z, 0.0)
    rows = z.shape[0]
    row_id = i * rows + lax.broadcasted_iota(jnp.int32, (rows, 1), 0)
    z = jnp.where(row_id < N, z, 0.0)
    part = jnp.sum(z, axis=0, keepdims=True)

    @pl.when(i == 0)
    def _():
        acc_ref[:] = jnp.zeros_like(acc_ref)

    acc_ref[:] += part

    @pl.when(i == pl.num_programs(0) - 1)
    def _():
        out_ref[:] = acc_ref[:]


def _graph_mlp(num, den, skip, wl_t, bl, wl1_t, bl1):
    rows = 512
    grid = (NPAD // rows,)
    out = pl.pallas_call(
        _mlp_body,
        grid=grid,
        in_specs=[
            pl.BlockSpec((NC, rows, HID), lambda i: (0, i, 0)),
            pl.BlockSpec((NC, rows, 16), lambda i: (0, i, 0)),
            pl.BlockSpec((rows, HID), lambda i: (i, 0)),
            pl.BlockSpec((HID, FF), lambda i: (0, 0)),
            pl.BlockSpec((1, FF), lambda i: (0, 0)),
            pl.BlockSpec((FF, D), lambda i: (0, 0)),
            pl.BlockSpec((1, D), lambda i: (0, 0)),
        ],
        out_specs=pl.BlockSpec((1, D), lambda i: (0, 0)),
        out_shape=jax.ShapeDtypeStruct((1, D), jnp.float32),
        scratch_shapes=[pltpu.VMEM((1, D), jnp.float32)],
    )(num, den, skip, wl_t, bl, wl1_t, bl1)
    return out


def _text_body(x_ref, wt_ref, bt_ref, wt1_ref, bt1_ref, out_ref):
    t = jnp.dot(x_ref[:], wt_ref[:], preferred_element_type=jnp.float32)
    t = jnp.maximum(t + bt_ref[:], 0.0)
    o = jnp.dot(t, wt1_ref[:], preferred_element_type=jnp.float32)
    out_ref[:] = jnp.maximum(o + bt1_ref[:], 0.0)


def _text_mlp(x_text, wt_t, bt, wt1_t, bt1):
    rows = 512
    grid = (B // rows,)
    return pl.pallas_call(
        _text_body,
        grid=grid,
        in_specs=[
            pl.BlockSpec((rows, D), lambda i: (i, 0)),
            pl.BlockSpec((D, HID), lambda i: (0, 0)),
            pl.BlockSpec((1, HID), lambda i: (0, 0)),
            pl.BlockSpec((HID, D), lambda i: (0, 0)),
            pl.BlockSpec((1, D), lambda i: (0, 0)),
        ],
        out_specs=pl.BlockSpec((rows, D), lambda i: (i, 0)),
        out_shape=jax.ShapeDtypeStruct((B, D), jnp.float32),
    )(x_text, wt_t, bt, wt1_t, bt1)


@jax.jit
def kernel(x_text, x_graph, edge_index, edge_attr, place_node,
           Wq, bq, Wk, bk, Wv, bv, Ws, bs,
           Wl, bl, Wl1, bl1, Wt, bt, Wt1, bt1):
    del edge_attr, place_node
    w_cat_t = jnp.concatenate([Wq, Wk, Wv, Ws], axis=0).T
    b_cat = jnp.concatenate([bq, bk, bv, bs]).reshape(1, 4 * HID)

    xg_pad = jnp.pad(x_graph, ((0, NPAD - N), (0, 0)))
    q, k, v, skip = _qkvs_projection(xg_pad, w_cat_t, b_cat)

    src = edge_index[0]
    dst = edge_index[1]
    num, den = _sc_edge_aggregate(q, k, v, src, dst)

    xt = _text_mlp(x_text, Wt.T, bt.reshape(1, HID), Wt1.T, bt1.reshape(1, D))

    xg = _graph_mlp(num, den, skip, Wl.T, bl.reshape(1, FF),
                    Wl1.T, bl1.reshape(1, D))
    return (xt, xg.reshape(D))


# speedup vs baseline: 12.7620x; 1.0142x over previous
import jax
import jax.numpy as jnp
from jax import lax
from jax.experimental import pallas as pl
from jax.experimental.pallas import tpu as pltpu
from jax.experimental.pallas import tpu_sc as plsc

D = 1536
HID = 128
N = 10000
E = 320000
B = 4096
FF = 4096

NC = 2
NS = 16
NW = NC * NS
EPW = E // NW
C = 40
UNR = 8
CHUNKS = EPW // C
NPAD = 10240
RPT = NPAD // NS
ACC_W = HID + 16
SCALE = 1.0 / (HID ** 0.5)


def _sc_edge_body(q_hbm, k_hbm, v_hbm, src_hbm, dst_hbm, num_hbm, den_hbm,
                  src_i, dst_i, dst_sc, q_rows, k_rows, v_rows, den_rows,
                  red_v, acc_sh, den_sh,
                  isem0, isem1, gsem0, gsem1, ssem0, ssem1):
    c = lax.axis_index("c")
    s = lax.axis_index("s")
    wid = c * NS + s
    ebase = wid * EPW

    isems = (isem0, isem1)
    gsems = (gsem0, gsem1)
    ssems = (ssem0, ssem1)

    zero16 = jnp.zeros((16,), jnp.float32)

    def zero_row(e, carry):
        for j in range(HID // 16):
            v_rows[0, e, pl.ds(16 * j, 16)] = zero16
        den_rows[0, e, pl.ds(0, 16)] = zero16
        return carry

    lax.fori_loop(0, C, zero_row, 0)

    for off in range(0, RPT, C):
        n = min(C, RPT - off)
        pltpu.sync_copy(v_rows.at[0, pl.ds(0, n)],
                        acc_sh.at[pl.ds(s * RPT + off, n)])
        pltpu.sync_copy(den_rows.at[0, pl.ds(0, n)],
                        den_sh.at[pl.ds(s * RPT + off, n)])
    plsc.subcore_barrier()

    lane = lax.iota(jnp.int32, 16)
    den_mask = jnp.where(lane == 0, 1.0, 0.0).astype(jnp.float32)

    def issue_idx(t, b):
        pltpu.async_copy(src_hbm.at[pl.ds(ebase + t * C, C)],
                         src_i.at[b], isems[b])
        pltpu.async_copy(dst_hbm.at[pl.ds(ebase + t * C, C)],
                         dst_i.at[b], isems[b])

    def wait_idx(b):
        pltpu.make_async_copy(src_hbm.at[pl.ds(0, C)], src_i.at[b],
                              isems[b]).wait()
        pltpu.make_async_copy(dst_hbm.at[pl.ds(0, C)], dst_i.at[b],
                              isems[b]).wait()

    def issue_gathers(b):
        pltpu.async_copy(q_hbm.at[dst_i.at[b]], q_rows.at[b], gsems[b])
        pltpu.async_copy(k_hbm.at[src_i.at[b]], k_rows.at[b], gsems[b])
        pltpu.async_copy(v_hbm.at[src_i.at[b]], v_rows.at[b], gsems[b])

    def wait_gathers(b):
        pltpu.make_async_copy(q_hbm.at[pl.ds(0, C)], q_rows.at[b],
                              gsems[b]).wait()
        pltpu.make_async_copy(k_hbm.at[pl.ds(0, C)], k_rows.at[b],
                              gsems[b]).wait()
        pltpu.make_async_copy(v_hbm.at[pl.ds(0, C)], v_rows.at[b],
                              gsems[b]).wait()

    def copy_idx(b):
        for off in (0, 16, C - 16):
            dst_sc[b, pl.ds(off, 16)] = dst_i[b, pl.ds(off, 16)]

    def issue_scatter(b):
        pltpu.async_copy(v_rows.at[b], acc_sh.at[dst_sc.at[b]], ssems[b],
                         add=True)
        pltpu.async_copy(den_rows.at[b], den_sh.at[dst_sc.at[b]], ssems[b],
                         add=True)

    def wait_scatter(b):
        pltpu.make_async_copy(v_rows.at[b], acc_sh.at[pl.ds(0, C)],
                              ssems[b]).wait()
        pltpu.make_async_copy(den_rows.at[b], den_sh.at[pl.ds(0, C)],
                              ssems[b]).wait()

    def compute_chunk(b):
        def edge_group(g, carry):
            e0 = g * UNR
            accs = []
            for u in range(UNR):
                e = e0 + u
                acc = q_rows[b, e, pl.ds(0, 16)] * k_rows[b, e, pl.ds(0, 16)]
                for j in range(1, HID // 16):
                    acc = acc + (q_rows[b, e, pl.ds(16 * j, 16)]
                                 * k_rows[b, e, pl.ds(16 * j, 16)])
                accs.append(acc)
            for sh in (8, 4, 2, 1):
                for u in range(UNR):
                    red_v[u, pl.ds(0, 16)] = accs[u]
                accs = [accs[u] + plsc.load_gather(red_v.at[u], [lane ^ sh])
                        for u in range(UNR)]
            ws = [jnp.exp(accs[u] * SCALE) for u in range(UNR)]
            for u in range(UNR):
                e = e0 + u
                for j in range(HID // 16):
                    sl = pl.ds(16 * j, 16)
                    v_rows[b, e, sl] = v_rows[b, e, sl] * ws[u]
                den_rows[b, e, pl.ds(0, 16)] = ws[u] * den_mask
            return carry

        lax.fori_loop(0, C // UNR, edge_group, 0)

    issue_idx(0, 0)
    issue_idx(1, 1)
    wait_idx(0)
    issue_gathers(0)

    def pipe_body(i, carry):
        t0 = 2 * i
        t1 = 2 * i + 1
        wait_idx(1)

        @pl.when(t0 > 0)
        def _():
            wait_scatter(1)

        issue_gathers(1)
        wait_gathers(0)
        copy_idx(0)

        @pl.when(t0 + 2 < CHUNKS)
        def _():
            issue_idx(t0 + 2, 0)

        compute_chunk(0)
        issue_scatter(0)

        @pl.when(t1 + 1 < CHUNKS)
        def _():
            wait_idx(0)
            wait_scatter(0)
            issue_gathers(0)

        wait_gathers(1)
        copy_idx(1)

        @pl.when(t1 + 2 < CHUNKS)
        def _():
            issue_idx(t1 + 2, 1)

        compute_chunk(1)
        issue_scatter(1)
        return carry

    lax.fori_loop(0, CHUNKS // 2, pipe_body, 0)
    wait_scatter(0)
    wait_scatter(1)
    plsc.subcore_barrier()
    pltpu.sync_copy(acc_sh.at[pl.ds(s * RPT, RPT)],
                    num_hbm.at[c, pl.ds(s * RPT, RPT)])
    pltpu.sync_copy(den_sh.at[pl.ds(s * RPT, RPT)],
                    den_hbm.at[c, pl.ds(s * RPT, RPT)])


def _sc_edge_aggregate(q, k, v, src, dst):
    mesh = plsc.VectorSubcoreMesh(core_axis_name="c", subcore_axis_name="s",
                                  num_cores=NC, num_subcores=NS)
    kern = pl.kernel(
        _sc_edge_body,
        out_type=(
            jax.ShapeDtypeStruct((NC, NPAD, HID), jnp.float32),
            jax.ShapeDtypeStruct((NC, NPAD, 16), jnp.float32),
        ),
        mesh=mesh,
        compiler_params=pltpu.CompilerParams(needs_layout_passes=False,
                                             use_tc_tiling_on_sc=False),
        scratch_types=[
            pltpu.VMEM((2, C), jnp.int32),
            pltpu.VMEM((2, C), jnp.int32),
            pltpu.VMEM((2, C), jnp.int32),
            pltpu.VMEM((2, C, HID), jnp.float32),
            pltpu.VMEM((2, C, HID), jnp.float32),
            pltpu.VMEM((2, C, HID), jnp.float32),
            pltpu.VMEM((2, C, 16), jnp.float32),
            pltpu.VMEM((UNR, 16), jnp.float32),
            pltpu.VMEM_SHARED((NPAD, HID), jnp.float32),
            pltpu.VMEM_SHARED((NPAD, 16), jnp.float32),
            pltpu.SemaphoreType.DMA,
            pltpu.SemaphoreType.DMA,
            pltpu.SemaphoreType.DMA,
            pltpu.SemaphoreType.DMA,
            pltpu.SemaphoreType.DMA,
            pltpu.SemaphoreType.DMA,
        ],
    )
    return kern(q, k, v, src, dst)


def _proj_body(x_ref, w_ref, b_ref, q_ref, k_ref, v_ref, skip_ref):
    y = jnp.dot(x_ref[:], w_ref[:], preferred_element_type=jnp.float32)
    y = y + b_ref[:]
    q_ref[:] = y[:, :HID]
    k_ref[:] = y[:, HID:2 * HID]
    v_ref[:] = y[:, 2 * HID:3 * HID]
    skip_ref[:] = y[:, 3 * HID:]


def _qkvs_projection(x_graph, w_cat_t, b_cat):
    rows = 1024
    grid = (NPAD // rows,)
    return pl.pallas_call(
        _proj_body,
        grid=grid,
        in_specs=[
            pl.BlockSpec((rows, D), lambda i: (i, 0)),
            pl.BlockSpec((D, 4 * HID), lambda i: (0, 0)),
            pl.BlockSpec((1, 4 * HID), lambda i: (0, 0)),
        ],
        out_specs=[
            pl.BlockSpec((rows, HID), lambda i: (i, 0)),
            pl.BlockSpec((rows, HID), lambda i: (i, 0)),
            pl.BlockSpec((rows, HID), lambda i: (i, 0)),
            pl.BlockSpec((rows, HID), lambda i: (i, 0)),
        ],
        out_shape=[
            jax.ShapeDtypeStruct((NPAD, HID), jnp.float32),
            jax.ShapeDtypeStruct((NPAD, HID), jnp.float32),
            jax.ShapeDtypeStruct((NPAD, HID), jnp.float32),
            jax.ShapeDtypeStruct((NPAD, HID), jnp.float32),
        ],
    )(x_graph, w_cat_t, b_cat)


def _mlp_body(num_ref, den_ref, skip_ref, wl_ref, bl_ref, wl1_ref, bl1_ref,
              out_ref, acc_ref):
    i = pl.program_id(0)
    num = num_ref[0] + num_ref[1]
    den = den_ref[0, :, 0:1] + den_ref[1, :, 0:1]
    h = jnp.maximum(num / (den + 1e-16) + skip_ref[:], 0.0)
    y = jnp.dot(h, wl_ref[:], preferred_element_type=jnp.float32) + bl_ref[:]
    y = jnp.maximum(y, 0.0)
    z = jnp.dot(y, wl1_ref[:], preferred_element_type=jnp.float32) + bl1_ref[:]
    z = jnp.maximum(z, 0.0)
    rows = z.shape[0]
    row_id = i * rows + lax.broadcasted_iota(jnp.int32, (rows, 1), 0)
    z = jnp.where(row_id < N, z, 0.0)
    part = jnp.sum(z, axis=0, keepdims=True)

    @pl.when(i == 0)
    def _():
        acc_ref[:] = jnp.zeros_like(acc_ref)

    acc_ref[:] += part

    @pl.when(i == pl.num_programs(0) - 1)
    def _():
        out_ref[:] = acc_ref[:]


def _graph_mlp(num, den, skip, wl_t, bl, wl1_t, bl1):
    rows = 512
    grid = (NPAD // rows,)
    out = pl.pallas_call(
        _mlp_body,
        grid=grid,
        in_specs=[
            pl.BlockSpec((NC, rows, HID), lambda i: (0, i, 0)),
            pl.BlockSpec((NC, rows, 16), lambda i: (0, i, 0)),
            pl.BlockSpec((rows, HID), lambda i: (i, 0)),
            pl.BlockSpec((HID, FF), lambda i: (0, 0)),
            pl.BlockSpec((1, FF), lambda i: (0, 0)),
            pl.BlockSpec((FF, D), lambda i: (0, 0)),
            pl.BlockSpec((1, D), lambda i: (0, 0)),
        ],
        out_specs=pl.BlockSpec((1, D), lambda i: (0, 0)),
        out_shape=jax.ShapeDtypeStruct((1, D), jnp.float32),
        scratch_shapes=[pltpu.VMEM((1, D), jnp.float32)],
    )(num, den, skip, wl_t, bl, wl1_t, bl1)
    return out


def _text_body(x_ref, wt_ref, bt_ref, wt1_ref, bt1_ref, out_ref):
    t = jnp.dot(x_ref[:], wt_ref[:], preferred_element_type=jnp.float32)
    t = jnp.maximum(t + bt_ref[:], 0.0)
    o = jnp.dot(t, wt1_ref[:], preferred_element_type=jnp.float32)
    out_ref[:] = jnp.maximum(o + bt1_ref[:], 0.0)


def _text_mlp(x_text, wt_t, bt, wt1_t, bt1):
    rows = 512
    grid = (B // rows,)
    return pl.pallas_call(
        _text_body,
        grid=grid,
        in_specs=[
            pl.BlockSpec((rows, D), lambda i: (i, 0)),
            pl.BlockSpec((D, HID), lambda i: (0, 0)),
            pl.BlockSpec((1, HID), lambda i: (0, 0)),
            pl.BlockSpec((HID, D), lambda i: (0, 0)),
            pl.BlockSpec((1, D), lambda i: (0, 0)),
        ],
        out_specs=pl.BlockSpec((rows, D), lambda i: (i, 0)),
        out_shape=jax.ShapeDtypeStruct((B, D), jnp.float32),
    )(x_text, wt_t, bt, wt1_t, bt1)


@jax.jit
def kernel(x_text, x_graph, edge_index, edge_attr, place_node,
           Wq, bq, Wk, bk, Wv, bv, Ws, bs,
           Wl, bl, Wl1, bl1, Wt, bt, Wt1, bt1):
    del edge_attr, place_node
    w_cat_t = jnp.concatenate([Wq, Wk, Wv, Ws], axis=0).T
    b_cat = jnp.concatenate([bq, bk, bv, bs]).reshape(1, 4 * HID)

    xg_pad = jnp.pad(x_graph, ((0, NPAD - N), (0, 0)))
    q, k, v, skip = _qkvs_projection(xg_pad, w_cat_t, b_cat)

    src = edge_index[0]
    dst = edge_index[1]
    num, den = _sc_edge_aggregate(q, k, v, src, dst)

    xt = _text_mlp(x_text, Wt.T, bt.reshape(1, HID), Wt1.T, bt1.reshape(1, D))

    xg = _graph_mlp(num, den, skip, Wl.T, bl.reshape(1, FF),
                    Wl1.T, bl1.reshape(1, D))
    return (xt, xg.reshape(D))
